# baseline (speedup 1.0000x reference)
import jax
import jax.numpy as jnp
from jax.experimental import pallas as pl
from jax.experimental.pallas import tpu as pltpu

_CHUNK_ROWS = 1024
_N_BUF = 4


def _copy_kernel(flat_ref, rs_ref, flat_out, rs_out, bufs, in_sems, out_sems,
                 rs_sem):
    n_rows = flat_ref.shape[0]
    n_chunks = n_rows // _CHUNK_ROWS

    rs_in = pltpu.make_async_copy(rs_ref, rs_out, rs_sem)
    rs_in.start()

    def make_in(c):
        src = flat_ref.at[pl.ds(c * _CHUNK_ROWS, _CHUNK_ROWS), :]
        return pltpu.make_async_copy(src, bufs.at[c % _N_BUF],
                                     in_sems.at[c % _N_BUF])

    def make_out(c):
        dst = flat_out.at[pl.ds(c * _CHUNK_ROWS, _CHUNK_ROWS), :]
        return pltpu.make_async_copy(bufs.at[c % _N_BUF], dst,
                                     out_sems.at[c % _N_BUF])

    lookahead = _N_BUF - 1
    in_copies = [None] * n_chunks
    out_copies = [None] * n_chunks
    for c in range(min(lookahead, n_chunks)):
        in_copies[c] = make_in(c)
        in_copies[c].start()
    for c in range(n_chunks):
        la = c + lookahead
        if la < n_chunks:
            if la >= _N_BUF:
                out_copies[la - _N_BUF].wait()
            in_copies[la] = make_in(la)
            in_copies[la].start()
        in_copies[c].wait()
        out_copies[c] = make_out(c)
        out_copies[c].start()
    for c in range(max(0, n_chunks - _N_BUF), n_chunks):
        out_copies[c].wait()
    rs_in.wait()


def kernel(flat, row_splits):
    n_rows, n_feat = flat.shape
    return pl.pallas_call(
        _copy_kernel,
        out_shape=(
            jax.ShapeDtypeStruct(flat.shape, flat.dtype),
            jax.ShapeDtypeStruct(row_splits.shape, row_splits.dtype),
        ),
        in_specs=[
            pl.BlockSpec(memory_space=pltpu.MemorySpace.HBM),
            pl.BlockSpec(memory_space=pltpu.MemorySpace.HBM),
        ],
        out_specs=(
            pl.BlockSpec(memory_space=pltpu.MemorySpace.HBM),
            pl.BlockSpec(memory_space=pltpu.MemorySpace.HBM),
        ),
        scratch_shapes=[
            pltpu.VMEM((_N_BUF, _CHUNK_ROWS, n_feat), jnp.float32),
            pltpu.SemaphoreType.DMA((_N_BUF,)),
            pltpu.SemaphoreType.DMA((_N_BUF,)),
            pltpu.SemaphoreType.DMA,
        ],
    )(flat, row_splits)


# speedup vs baseline: 1.1335x; 1.1335x over previous
import jax
import jax.numpy as jnp
from jax.experimental import pallas as pl
from jax.experimental.pallas import tpu as pltpu

_BLOCK = 4096


def _copy_kernel(flat_ref, rs_ref, flat_out, rs_out):
    flat_out[...] = flat_ref[...]

    @pl.when(pl.program_id(0) == 0)
    def _():
        for i in range(rs_ref.shape[0]):
            rs_out[i] = rs_ref[i]


def kernel(flat, row_splits):
    n_rows, n_feat = flat.shape
    grid = (n_rows // _BLOCK,)
    return pl.pallas_call(
        _copy_kernel,
        grid=grid,
        out_shape=(
            jax.ShapeDtypeStruct(flat.shape, flat.dtype),
            jax.ShapeDtypeStruct(row_splits.shape, row_splits.dtype),
        ),
        in_specs=[
            pl.BlockSpec((_BLOCK, n_feat), lambda i: (i, 0)),
            pl.BlockSpec(memory_space=pltpu.MemorySpace.SMEM),
        ],
        out_specs=(
            pl.BlockSpec((_BLOCK, n_feat), lambda i: (i, 0)),
            pl.BlockSpec(memory_space=pltpu.MemorySpace.SMEM),
        ),
    )(flat, row_splits)
